# baseline (device time: 24331 ns/iter reference)
import jax
import jax.numpy as jnp
from jax import lax
from jax.experimental import pallas as pl
from jax.experimental.pallas import tpu as pltpu

X = 2
Y = 4
Z = 4
M_OUT = 128
D = 512
N = 2048
BW = N // (X * Z)


def kernel(x, dy):
    def body(x_ref, dy_ref, out_ref, part_ref, rs_buf, zbuf, xbuf,
             rs_send, rs_recv, z_send, z_recv, x_send, x_recv):
        my_x = lax.axis_index("x")
        my_y = lax.axis_index("y")
        my_z = lax.axis_index("z")
        peer_x = 1 - my_x

        barrier_sem = pltpu.get_barrier_semaphore()
        for d in range(1, Y):
            pl.semaphore_signal(
                barrier_sem, inc=1,
                device_id=(my_x, lax.rem(my_y + d, Y), my_z),
                device_id_type=pl.DeviceIdType.MESH,
            )
        for d in range(1, Z):
            pl.semaphore_signal(
                barrier_sem, inc=1,
                device_id=(my_x, my_y, lax.rem(my_z + d, Z)),
                device_id_type=pl.DeviceIdType.MESH,
            )
        pl.semaphore_signal(
            barrier_sem, inc=1,
            device_id=(peer_x, my_y, my_z),
            device_id_type=pl.DeviceIdType.MESH,
        )
        pl.semaphore_wait(barrier_sem, (Y - 1) + (Z - 1) + 1)

        def col0(xb, zb):
            return (2 * zb + xb) * BW

        part = lax.dot_general(
            x_ref[...].astype(jnp.bfloat16),
            dy_ref[:, pl.ds(col0(my_x, my_z), BW)].astype(jnp.bfloat16),
            dimension_numbers=(((0,), (0,)), ((), ())),
            preferred_element_type=jnp.float32,
        )
        part_ref[...] = part.astype(jnp.bfloat16)

        def rows(j):
            return pl.ds(j * M_OUT, M_OUT)

        rs_rdmas = []
        for d in range(Y - 1, 0, -1):
            tgt = lax.rem(my_y + d, Y)
            rdma = pltpu.make_async_remote_copy(
                src_ref=part_ref.at[rows(tgt), :],
                dst_ref=rs_buf.at[d - 1],
                send_sem=rs_send.at[d - 1],
                recv_sem=rs_recv.at[d - 1],
                device_id=(my_x, tgt, my_z),
                device_id_type=pl.DeviceIdType.MESH,
            )
            rdma.start()
            rs_rdmas.append(rdma)
        for rdma in rs_rdmas:
            rdma.wait_recv()
        acc = (
            part_ref[rows(my_y), :].astype(jnp.float32)
            + rs_buf[0].astype(jnp.float32)
            + rs_buf[1].astype(jnp.float32)
            + rs_buf[2].astype(jnp.float32)
        )
        out_ref[:, pl.ds(col0(my_x, my_z), BW)] = acc
        zbuf[Z - 1, :, :] = acc.astype(jnp.bfloat16)

        x_own = pltpu.make_async_remote_copy(
            src_ref=zbuf.at[Z - 1],
            dst_ref=xbuf.at[0],
            send_sem=x_send.at[0],
            recv_sem=x_recv.at[0],
            device_id=(peer_x, my_y, my_z),
            device_id_type=pl.DeviceIdType.MESH,
        )
        x_own.start()

        z_rdmas = [None] * (Z - 1)
        for d in range(Z - 1, 0, -1):
            tgt = lax.rem(my_z + d, Z)
            rdma = pltpu.make_async_remote_copy(
                src_ref=zbuf.at[Z - 1],
                dst_ref=zbuf.at[d - 1],
                send_sem=z_send.at[d - 1],
                recv_sem=z_recv.at[d - 1],
                device_id=(my_x, my_y, tgt),
                device_id_type=pl.DeviceIdType.MESH,
            )
            rdma.start()
            z_rdmas[d - 1] = rdma

        x_fwd = [None] * (Z - 1)
        for d in range(1, Z):
            z_rdmas[d - 1].wait_recv()
            src_z = lax.rem(my_z + Z - d, Z)
            out_ref[:, pl.ds(col0(my_x, src_z), BW)] = (
                zbuf[d - 1].astype(jnp.float32)
            )
            fwd = pltpu.make_async_remote_copy(
                src_ref=zbuf.at[d - 1],
                dst_ref=xbuf.at[d],
                send_sem=x_send.at[d],
                recv_sem=x_recv.at[d],
                device_id=(peer_x, my_y, my_z),
                device_id_type=pl.DeviceIdType.MESH,
            )
            fwd.start()
            x_fwd[d - 1] = fwd

        x_own.wait_recv()
        out_ref[:, pl.ds(col0(peer_x, my_z), BW)] = xbuf[0].astype(jnp.float32)
        for d in range(1, Z):
            x_fwd[d - 1].wait_recv()
            src_z = lax.rem(my_z + Z - d, Z)
            out_ref[:, pl.ds(col0(peer_x, src_z), BW)] = (
                xbuf[d].astype(jnp.float32)
            )

        for rdma in rs_rdmas + z_rdmas + [x_own] + x_fwd:
            rdma.wait_send()

    return pl.pallas_call(
        body,
        out_shape=jax.ShapeDtypeStruct((M_OUT, N), jnp.float32),
        in_specs=[
            pl.BlockSpec(memory_space=pltpu.VMEM),
            pl.BlockSpec(memory_space=pltpu.VMEM),
        ],
        out_specs=pl.BlockSpec(memory_space=pltpu.VMEM),
        scratch_shapes=[
            pltpu.VMEM((D, BW), jnp.bfloat16),
            pltpu.VMEM((Y - 1, M_OUT, BW), jnp.bfloat16),
            pltpu.VMEM((Z, M_OUT, BW), jnp.bfloat16),
            pltpu.VMEM((Z, M_OUT, BW), jnp.bfloat16),
            pltpu.SemaphoreType.DMA((Y - 1,)),
            pltpu.SemaphoreType.DMA((Y - 1,)),
            pltpu.SemaphoreType.DMA((Z - 1,)),
            pltpu.SemaphoreType.DMA((Z - 1,)),
            pltpu.SemaphoreType.DMA((Z,)),
            pltpu.SemaphoreType.DMA((Z,)),
        ],
        compiler_params=pltpu.CompilerParams(collective_id=0),
    )(x, dy)


# device time: 23874 ns/iter; 1.0191x vs baseline; 1.0191x over previous
import jax
import jax.numpy as jnp
from jax import lax
from jax.experimental import pallas as pl
from jax.experimental.pallas import tpu as pltpu

X = 2
Y = 4
Z = 4
G = X * Z
M_OUT = 128
D = 512
N = 2048
BW = N // G


def kernel(x, dy):
    def body(x_ref, dy_ref, out_ref, part_ref, rs_buf, gbuf,
             rs_send, rs_recv, g_send, g_recv):
        my_x = lax.axis_index("x")
        my_y = lax.axis_index("y")
        my_z = lax.axis_index("z")
        my_g = 2 * my_z + my_x

        barrier_sem = pltpu.get_barrier_semaphore()
        for d in range(1, Y):
            pl.semaphore_signal(
                barrier_sem, inc=1,
                device_id=(my_x, lax.rem(my_y + d, Y), my_z),
                device_id_type=pl.DeviceIdType.MESH,
            )
        for o in range(1, G):
            tgt_g = lax.rem(my_g + o, G)
            pl.semaphore_signal(
                barrier_sem, inc=1,
                device_id=(lax.rem(tgt_g, 2), my_y, tgt_g // 2),
                device_id_type=pl.DeviceIdType.MESH,
            )
        pl.semaphore_wait(barrier_sem, (Y - 1) + (G - 1))

        part = lax.dot_general(
            x_ref[...].astype(jnp.bfloat16),
            dy_ref[:, pl.ds(my_g * BW, BW)].astype(jnp.bfloat16),
            dimension_numbers=(((0,), (0,)), ((), ())),
            preferred_element_type=jnp.float32,
        )
        part_ref[...] = part.astype(jnp.bfloat16)

        def rows(j):
            return pl.ds(j * M_OUT, M_OUT)

        rs_rdmas = []
        for d in range(Y - 1, 0, -1):
            tgt = lax.rem(my_y + d, Y)
            rdma = pltpu.make_async_remote_copy(
                src_ref=part_ref.at[rows(tgt), :],
                dst_ref=rs_buf.at[d - 1],
                send_sem=rs_send.at[d - 1],
                recv_sem=rs_recv.at[d - 1],
                device_id=(my_x, tgt, my_z),
                device_id_type=pl.DeviceIdType.MESH,
            )
            rdma.start()
            rs_rdmas.append(rdma)
        for rdma in rs_rdmas:
            rdma.wait_recv()
        acc = (
            part_ref[rows(my_y), :].astype(jnp.float32)
            + rs_buf[0].astype(jnp.float32)
            + rs_buf[1].astype(jnp.float32)
            + rs_buf[2].astype(jnp.float32)
        )
        out_ref[:, pl.ds(my_g * BW, BW)] = acc
        gbuf[G - 1, :, :] = acc.astype(jnp.bfloat16)

        g_rdmas = [None] * (G - 1)
        for o in range(G - 1, 0, -1):
            tgt_g = lax.rem(my_g + o, G)
            rdma = pltpu.make_async_remote_copy(
                src_ref=gbuf.at[G - 1],
                dst_ref=gbuf.at[o - 1],
                send_sem=g_send.at[o - 1],
                recv_sem=g_recv.at[o - 1],
                device_id=(lax.rem(tgt_g, 2), my_y, tgt_g // 2),
                device_id_type=pl.DeviceIdType.MESH,
            )
            rdma.start()
            g_rdmas[o - 1] = rdma

        for o in range(1, G):
            g_rdmas[o - 1].wait_recv()
            src_g = lax.rem(my_g + G - o, G)
            out_ref[:, pl.ds(src_g * BW, BW)] = gbuf[o - 1].astype(jnp.float32)

        for rdma in rs_rdmas + g_rdmas:
            rdma.wait_send()

    return pl.pallas_call(
        body,
        out_shape=jax.ShapeDtypeStruct((M_OUT, N), jnp.float32),
        in_specs=[
            pl.BlockSpec(memory_space=pltpu.VMEM),
            pl.BlockSpec(memory_space=pltpu.VMEM),
        ],
        out_specs=pl.BlockSpec(memory_space=pltpu.VMEM),
        scratch_shapes=[
            pltpu.VMEM((D, BW), jnp.bfloat16),
            pltpu.VMEM((Y - 1, M_OUT, BW), jnp.bfloat16),
            pltpu.VMEM((G, M_OUT, BW), jnp.bfloat16),
            pltpu.SemaphoreType.DMA((Y - 1,)),
            pltpu.SemaphoreType.DMA((Y - 1,)),
            pltpu.SemaphoreType.DMA((G - 1,)),
            pltpu.SemaphoreType.DMA((G - 1,)),
        ],
        compiler_params=pltpu.CompilerParams(collective_id=0),
    )(x, dy)
